# baseline (speedup 1.0000x reference)
import functools

import jax
import jax.numpy as jnp
from jax import lax
from jax.experimental import pallas as pl
from jax.experimental.pallas import tpu as pltpu
from jax.experimental.pallas import tpu_sc as plsc

D = 128
L = 16
_UNROLL = 8


def _node_dot_body(x_hbm, s_hbm, r_hbm, out_hbm,
                   s_v, r_v, xs_v, xr_v, o_v, sem_s, sem_r,
                   *, n_edges, chunk, num_workers):
    per_w = n_edges // num_workers
    n_chunks = per_w // chunk
    n_groups = chunk // L

    cid = lax.axis_index("c")
    sid = lax.axis_index("s")
    wid = sid * 2 + cid
    base = pl.multiple_of(wid * per_w, chunk)

    iota = lax.iota(jnp.int32, L)

    def chunk_body(c, _):
        off = pl.multiple_of(base + c * chunk, chunk)
        pltpu.sync_copy(s_hbm.at[pl.ds(off, chunk)], s_v)
        pltpu.sync_copy(r_hbm.at[pl.ds(off, chunk)], r_v)
        cps = pltpu.async_copy(x_hbm.at[s_v], xs_v, sem_s)
        cpr = pltpu.async_copy(x_hbm.at[r_v], xr_v, sem_r)
        cps.wait()
        cpr.wait()

        def group_body(g, _):
            row = g * L + iota

            def d_body(dd, carry):
                acc, col = carry
                for _j in range(_UNROLL):
                    a = plsc.load_gather(xs_v, [row, col])
                    b = plsc.load_gather(xr_v, [row, col])
                    acc = acc + a * b
                    col = (col + 1) & (D - 1)
                return acc, col

            acc0 = jnp.zeros((L,), jnp.float32)
            acc, _col = lax.fori_loop(0, D // _UNROLL, d_body, (acc0, iota))
            o_v[pl.ds(g * L, L)] = acc
            return 0

        lax.fori_loop(0, n_groups, group_body, 0)
        pltpu.sync_copy(o_v, out_hbm.at[pl.ds(off, chunk)])
        return 0

    lax.fori_loop(0, n_chunks, chunk_body, 0)


def kernel(x, senders, receivers):
    n_edges = senders.shape[0]
    info = plsc.get_sparse_core_info()
    nw = info.num_cores * info.num_subcores
    chunk = 400
    assert n_edges % (nw * chunk) == 0

    mesh = plsc.VectorSubcoreMesh(core_axis_name="c", subcore_axis_name="s")
    body = functools.partial(
        _node_dot_body, n_edges=n_edges, chunk=chunk, num_workers=nw)
    k = pl.kernel(
        body,
        out_type=jax.ShapeDtypeStruct((n_edges,), jnp.float32),
        mesh=mesh,
        scratch_types=[
            pltpu.VMEM((chunk,), jnp.int32),
            pltpu.VMEM((chunk,), jnp.int32),
            pltpu.VMEM((chunk, D), jnp.float32),
            pltpu.VMEM((chunk, D), jnp.float32),
            pltpu.VMEM((chunk,), jnp.float32),
            pltpu.SemaphoreType.DMA,
            pltpu.SemaphoreType.DMA,
        ],
        compiler_params=pltpu.CompilerParams(needs_layout_passes=False),
    )
    return k(x, senders.astype(jnp.int32), receivers.astype(jnp.int32))


# speedup vs baseline: 1.6226x; 1.6226x over previous
import functools

import jax
import jax.numpy as jnp
from jax import lax
from jax.experimental import pallas as pl
from jax.experimental.pallas import tpu as pltpu
from jax.experimental.pallas import tpu_sc as plsc

D = 128
L = 16
_UNROLL = 8


def _node_dot_body(x_hbm, s_hbm, r_hbm, out_hbm,
                   s_all, r_all, xs_v, xr_v, o_all, sem_s, sem_r,
                   *, n_edges, chunk, num_workers):
    per_w = n_edges // num_workers
    n_chunks = per_w // chunk
    n_groups = chunk // L

    cid = lax.axis_index("c")
    sid = lax.axis_index("s")
    wid = sid * 2 + cid
    base = pl.multiple_of(wid * per_w, chunk)

    iota = lax.iota(jnp.int32, L)

    pltpu.sync_copy(s_hbm.at[pl.ds(base, per_w)], s_all)
    pltpu.sync_copy(r_hbm.at[pl.ds(base, per_w)], r_all)

    def start(c, p):
        sl = pl.ds(pl.multiple_of(c * chunk, chunk), chunk)
        pltpu.async_copy(x_hbm.at[s_all.at[sl]], xs_v.at[p], sem_s.at[p])
        pltpu.async_copy(x_hbm.at[r_all.at[sl]], xr_v.at[p], sem_r.at[p])

    def wait(p):
        pltpu.make_async_copy(x_hbm.at[s_all.at[pl.ds(0, chunk)]],
                              xs_v.at[p], sem_s.at[p]).wait()
        pltpu.make_async_copy(x_hbm.at[r_all.at[pl.ds(0, chunk)]],
                              xr_v.at[p], sem_r.at[p]).wait()

    def compute(c, p):
        obase = pl.multiple_of(c * chunk, chunk)

        def group_body(g, _):
            row = g * L + iota

            def d_body(dd, carry):
                acc, col = carry
                for _j in range(_UNROLL):
                    a = plsc.load_gather(xs_v.at[p], [row, col])
                    b = plsc.load_gather(xr_v.at[p], [row, col])
                    acc = acc + a * b
                    col = (col + 1) & (D - 1)
                return acc, col

            acc0 = jnp.zeros((L,), jnp.float32)
            acc, _col = lax.fori_loop(0, D // _UNROLL, d_body, (acc0, iota))
            o_all[pl.ds(obase + g * L, L)] = acc
            return 0

        lax.fori_loop(0, n_groups, group_body, 0)

    start(0, 0)
    def pair_body(g, _):
        c0 = g * 2
        start(c0 + 1, 1)
        wait(0)
        compute(c0, 0)
        start(c0 + 2, 0)
        wait(1)
        compute(c0 + 1, 1)
        return 0
    lax.fori_loop(0, (n_chunks - 1) // 2, pair_body, 0)
    wait(0)
    compute(n_chunks - 1, 0)

    pltpu.sync_copy(o_all, out_hbm.at[pl.ds(base, per_w)])


def kernel(x, senders, receivers):
    n_edges = senders.shape[0]
    info = plsc.get_sparse_core_info()
    nw = info.num_cores * info.num_subcores
    chunk = 80
    per_w = n_edges // nw
    assert n_edges % (nw * chunk) == 0 and (per_w // chunk) % 2 == 1

    mesh = plsc.VectorSubcoreMesh(core_axis_name="c", subcore_axis_name="s")
    body = functools.partial(
        _node_dot_body, n_edges=n_edges, chunk=chunk, num_workers=nw)
    k = pl.kernel(
        body,
        out_type=jax.ShapeDtypeStruct((n_edges,), jnp.float32),
        mesh=mesh,
        scratch_types=[
            pltpu.VMEM((per_w,), jnp.int32),
            pltpu.VMEM((per_w,), jnp.int32),
            pltpu.VMEM((2, chunk, D), jnp.float32),
            pltpu.VMEM((2, chunk, D), jnp.float32),
            pltpu.VMEM((per_w,), jnp.float32),
            pltpu.SemaphoreType.DMA((2,)),
            pltpu.SemaphoreType.DMA((2,)),
        ],
        compiler_params=pltpu.CompilerParams(needs_layout_passes=False),
    )
    return k(x, senders.astype(jnp.int32), receivers.astype(jnp.int32))


# speedup vs baseline: 1.6815x; 1.0363x over previous
import functools

import jax
import jax.numpy as jnp
from jax import lax
from jax.experimental import pallas as pl
from jax.experimental.pallas import tpu as pltpu
from jax.experimental.pallas import tpu_sc as plsc

D = 128
L = 16
_UNROLL = 8


def _node_dot_body(x_hbm, s_hbm, r_hbm, out_hbm,
                   s_all, r_all, xs_v, xr_v, o_all, sem_s, sem_r,
                   *, n_edges, chunk, num_workers):
    per_w = n_edges // num_workers
    n_chunks = per_w // chunk
    n_groups = chunk // L

    cid = lax.axis_index("c")
    sid = lax.axis_index("s")
    wid = sid * 2 + cid
    base = pl.multiple_of(wid * per_w, chunk)

    iota = lax.iota(jnp.int32, L)

    pltpu.sync_copy(s_hbm.at[pl.ds(base, per_w)], s_all)
    pltpu.sync_copy(r_hbm.at[pl.ds(base, per_w)], r_all)

    def start(c, p):
        sl = pl.ds(pl.multiple_of(c * chunk, chunk), chunk)
        pltpu.async_copy(x_hbm.at[s_all.at[sl]], xs_v.at[p], sem_s.at[p])
        pltpu.async_copy(x_hbm.at[r_all.at[sl]], xr_v.at[p], sem_r.at[p])

    def wait(p):
        pltpu.make_async_copy(x_hbm.at[s_all.at[pl.ds(0, chunk)]],
                              xs_v.at[p], sem_s.at[p]).wait()
        pltpu.make_async_copy(x_hbm.at[r_all.at[pl.ds(0, chunk)]],
                              xr_v.at[p], sem_r.at[p]).wait()

    def compute(c, p):
        obase = pl.multiple_of(c * chunk, chunk)
        xs_w = xs_v.at[p]
        xr_w = xr_v.at[p]
        dw = D // 2

        def group_body(g, _):
            row = g * L + iota

            def d_body(dd, carry):
                acc, col = carry
                for _j in range(_UNROLL):
                    a = plsc.load_gather(xs_w, [row, col])
                    b = plsc.load_gather(xr_w, [row, col])
                    a_lo, a_hi = plsc.unpack(
                        plsc.bitcast(a, jnp.bfloat16),
                        format=plsc.PackFormat.INTERLEAVED)
                    b_lo, b_hi = plsc.unpack(
                        plsc.bitcast(b, jnp.bfloat16),
                        format=plsc.PackFormat.INTERLEAVED)
                    acc = acc + a_lo * b_lo + a_hi * b_hi
                    col = (col + 1) & (dw - 1)
                return acc, col

            acc0 = jnp.zeros((L,), jnp.float32)
            acc, _col = lax.fori_loop(0, dw // _UNROLL, d_body, (acc0, iota))
            o_all[pl.ds(obase + g * L, L)] = acc
            return 0

        lax.fori_loop(0, n_groups, group_body, 0)

    start(0, 0)
    def pair_body(g, _):
        c0 = g * 2
        start(c0 + 1, 1)
        wait(0)
        compute(c0, 0)
        start(c0 + 2, 0)
        wait(1)
        compute(c0 + 1, 1)
        return 0
    lax.fori_loop(0, (n_chunks - 1) // 2, pair_body, 0)
    wait(0)
    compute(n_chunks - 1, 0)

    pltpu.sync_copy(o_all, out_hbm.at[pl.ds(base, per_w)])


def kernel(x, senders, receivers):
    n_edges = senders.shape[0]
    info = plsc.get_sparse_core_info()
    nw = info.num_cores * info.num_subcores
    chunk = 80
    per_w = n_edges // nw
    assert n_edges % (nw * chunk) == 0 and (per_w // chunk) % 2 == 1

    mesh = plsc.VectorSubcoreMesh(core_axis_name="c", subcore_axis_name="s")
    body = functools.partial(
        _node_dot_body, n_edges=n_edges, chunk=chunk, num_workers=nw)
    k = pl.kernel(
        body,
        out_type=jax.ShapeDtypeStruct((n_edges,), jnp.float32),
        mesh=mesh,
        scratch_types=[
            pltpu.VMEM((per_w,), jnp.int32),
            pltpu.VMEM((per_w,), jnp.int32),
            pltpu.VMEM((2, chunk, D // 2), jnp.float32),
            pltpu.VMEM((2, chunk, D // 2), jnp.float32),
            pltpu.VMEM((per_w,), jnp.float32),
            pltpu.SemaphoreType.DMA((2,)),
            pltpu.SemaphoreType.DMA((2,)),
        ],
        compiler_params=pltpu.CompilerParams(
            needs_layout_passes=False, use_tc_tiling_on_sc=False),
    )
    xb = x.astype(jnp.bfloat16).reshape(x.shape[0], D // 2, 2)
    xw = jax.lax.bitcast_convert_type(xb, jnp.float32)
    return k(xw, senders.astype(jnp.int32), receivers.astype(jnp.int32))


# speedup vs baseline: 1.8534x; 1.1022x over previous
import functools

import jax
import jax.numpy as jnp
from jax import lax
from jax.experimental import pallas as pl
from jax.experimental.pallas import tpu as pltpu
from jax.experimental.pallas import tpu_sc as plsc

D = 128
L = 16
_UNROLL = 8


def _node_dot_body(x_hbm, s_hbm, r_hbm, out_hbm,
                   s_all, r_all, xs_v, xr_v, o_all, sem_s, sem_r,
                   *, n_edges, chunk, num_workers):
    per_w = n_edges // num_workers
    n_chunks = per_w // chunk
    n_groups = chunk // L

    cid = lax.axis_index("c")
    sid = lax.axis_index("s")
    wid = sid * 2 + cid
    base = pl.multiple_of(wid * per_w, chunk)

    iota = lax.iota(jnp.int32, L)

    pltpu.sync_copy(s_hbm.at[pl.ds(base, per_w)], s_all)
    pltpu.sync_copy(r_hbm.at[pl.ds(base, per_w)], r_all)

    def start(c, p):
        sl = pl.ds(pl.multiple_of(c * chunk, chunk), chunk)
        pltpu.async_copy(x_hbm.at[s_all.at[sl]], xs_v.at[p], sem_s.at[p])
        pltpu.async_copy(x_hbm.at[r_all.at[sl]], xr_v.at[p], sem_r.at[p])

    def wait(p):
        pltpu.make_async_copy(x_hbm.at[s_all.at[pl.ds(0, chunk)]],
                              xs_v.at[p], sem_s.at[p]).wait()
        pltpu.make_async_copy(x_hbm.at[r_all.at[pl.ds(0, chunk)]],
                              xr_v.at[p], sem_r.at[p]).wait()

    def compute(c, p):
        obase = pl.multiple_of(c * chunk, chunk)
        xs_w = xs_v.at[p]
        xr_w = xr_v.at[p]
        dw = D // 2

        def group_body(g, _):
            row = g * L + iota

            def d_body(dd, carry):
                acc, acc2, col = carry
                for _j in range(_UNROLL):
                    a = plsc.load_gather(xs_w, [row, col])
                    b = plsc.load_gather(xr_w, [row, col])
                    m = (plsc.bitcast(a, jnp.bfloat16)
                         * plsc.bitcast(b, jnp.bfloat16))
                    m_lo, m_hi = plsc.unpack(
                        m, format=plsc.PackFormat.INTERLEAVED)
                    acc = acc + m_lo
                    acc2 = acc2 + m_hi
                    col = (col + 1) & (dw - 1)
                return acc, acc2, col

            acc0 = jnp.zeros((L,), jnp.float32)
            acc, acc2, _col = lax.fori_loop(
                0, dw // _UNROLL, d_body, (acc0, acc0, iota))
            o_all[pl.ds(obase + g * L, L)] = acc + acc2
            return 0

        lax.fori_loop(0, n_groups, group_body, 0)

    start(0, 0)
    def pair_body(g, _):
        c0 = g * 2
        start(c0 + 1, 1)
        wait(0)
        compute(c0, 0)
        start(c0 + 2, 0)
        wait(1)
        compute(c0 + 1, 1)
        return 0
    lax.fori_loop(0, (n_chunks - 1) // 2, pair_body, 0)
    wait(0)
    compute(n_chunks - 1, 0)

    pltpu.sync_copy(o_all, out_hbm.at[pl.ds(base, per_w)])


def kernel(x, senders, receivers):
    n_edges = senders.shape[0]
    info = plsc.get_sparse_core_info()
    nw = info.num_cores * info.num_subcores
    chunk = 80
    per_w = n_edges // nw
    assert n_edges % (nw * chunk) == 0 and (per_w // chunk) % 2 == 1

    mesh = plsc.VectorSubcoreMesh(core_axis_name="c", subcore_axis_name="s")
    body = functools.partial(
        _node_dot_body, n_edges=n_edges, chunk=chunk, num_workers=nw)
    k = pl.kernel(
        body,
        out_type=jax.ShapeDtypeStruct((n_edges,), jnp.float32),
        mesh=mesh,
        scratch_types=[
            pltpu.VMEM((per_w,), jnp.int32),
            pltpu.VMEM((per_w,), jnp.int32),
            pltpu.VMEM((2, chunk, D // 2), jnp.float32),
            pltpu.VMEM((2, chunk, D // 2), jnp.float32),
            pltpu.VMEM((per_w,), jnp.float32),
            pltpu.SemaphoreType.DMA((2,)),
            pltpu.SemaphoreType.DMA((2,)),
        ],
        compiler_params=pltpu.CompilerParams(
            needs_layout_passes=False, use_tc_tiling_on_sc=False),
    )
    xb = x.astype(jnp.bfloat16).reshape(x.shape[0], D // 2, 2)
    xw = jax.lax.bitcast_convert_type(xb, jnp.float32)
    return k(xw, senders.astype(jnp.int32), receivers.astype(jnp.int32))


# speedup vs baseline: 1.9238x; 1.0380x over previous
import functools

import jax
import jax.numpy as jnp
from jax import lax
from jax.experimental import pallas as pl
from jax.experimental.pallas import tpu as pltpu
from jax.experimental.pallas import tpu_sc as plsc

D = 128
L = 16
_UNROLL = 8


def _node_dot_body(x_hbm, s_hbm, r_hbm, out_hbm,
                   s_all, r_all, xs_v, xr_v, o_all, sem_s, sem_r,
                   *, n_edges, chunk, num_workers):
    per_w = n_edges // num_workers
    n_chunks = per_w // chunk
    n_groups = chunk // L

    cid = lax.axis_index("c")
    sid = lax.axis_index("s")
    wid = sid * 2 + cid
    base = pl.multiple_of(wid * per_w, chunk)

    iota = lax.iota(jnp.int32, L)

    pltpu.sync_copy(s_hbm.at[pl.ds(base, per_w)], s_all)
    pltpu.sync_copy(r_hbm.at[pl.ds(base, per_w)], r_all)

    def start(c, p):
        sl = pl.ds(pl.multiple_of(c * chunk, chunk), chunk)
        pltpu.async_copy(x_hbm.at[s_all.at[sl]], xs_v.at[p], sem_s.at[p])
        pltpu.async_copy(x_hbm.at[r_all.at[sl]], xr_v.at[p], sem_r.at[p])

    def wait(p):
        pltpu.make_async_copy(x_hbm.at[s_all.at[pl.ds(0, chunk)]],
                              xs_v.at[p], sem_s.at[p]).wait()
        pltpu.make_async_copy(x_hbm.at[r_all.at[pl.ds(0, chunk)]],
                              xr_v.at[p], sem_r.at[p]).wait()

    def compute(c, p):
        obase = pl.multiple_of(c * chunk, chunk)
        xs_w = xs_v.at[p]
        xr_w = xr_v.at[p]
        dw = D // 2

        def group_body(g, _):
            row = g * L + iota

            def d_body(dd, carry):
                acc, acc2, col = carry
                for _q in range(_UNROLL // 4):
                    mc = None
                    for _j in range(4):
                        a = plsc.load_gather(xs_w, [row, col])
                        b = plsc.load_gather(xr_w, [row, col])
                        m = (plsc.bitcast(a, jnp.bfloat16)
                             * plsc.bitcast(b, jnp.bfloat16))
                        mc = m if mc is None else mc + m
                        col = (col + 1) & (dw - 1)
                    m_lo, m_hi = plsc.unpack(
                        mc, format=plsc.PackFormat.INTERLEAVED)
                    acc = acc + m_lo
                    acc2 = acc2 + m_hi
                return acc, acc2, col

            acc0 = jnp.zeros((L,), jnp.float32)
            acc, acc2, _col = lax.fori_loop(
                0, dw // _UNROLL, d_body, (acc0, acc0, iota))
            o_all[pl.ds(obase + g * L, L)] = acc + acc2
            return 0

        lax.fori_loop(0, n_groups, group_body, 0)

    start(0, 0)
    def pair_body(g, _):
        c0 = g * 2
        start(c0 + 1, 1)
        wait(0)
        compute(c0, 0)
        start(c0 + 2, 0)
        wait(1)
        compute(c0 + 1, 1)
        return 0
    lax.fori_loop(0, (n_chunks - 1) // 2, pair_body, 0)
    wait(0)
    compute(n_chunks - 1, 0)

    pltpu.sync_copy(o_all, out_hbm.at[pl.ds(base, per_w)])


def kernel(x, senders, receivers):
    n_edges = senders.shape[0]
    info = plsc.get_sparse_core_info()
    nw = info.num_cores * info.num_subcores
    chunk = 80
    per_w = n_edges // nw
    assert n_edges % (nw * chunk) == 0 and (per_w // chunk) % 2 == 1

    mesh = plsc.VectorSubcoreMesh(core_axis_name="c", subcore_axis_name="s")
    body = functools.partial(
        _node_dot_body, n_edges=n_edges, chunk=chunk, num_workers=nw)
    k = pl.kernel(
        body,
        out_type=jax.ShapeDtypeStruct((n_edges,), jnp.float32),
        mesh=mesh,
        scratch_types=[
            pltpu.VMEM((per_w,), jnp.int32),
            pltpu.VMEM((per_w,), jnp.int32),
            pltpu.VMEM((2, chunk, D // 2), jnp.float32),
            pltpu.VMEM((2, chunk, D // 2), jnp.float32),
            pltpu.VMEM((per_w,), jnp.float32),
            pltpu.SemaphoreType.DMA((2,)),
            pltpu.SemaphoreType.DMA((2,)),
        ],
        compiler_params=pltpu.CompilerParams(
            needs_layout_passes=False, use_tc_tiling_on_sc=False),
    )
    xb = x.astype(jnp.bfloat16).reshape(x.shape[0], D // 2, 2)
    xw = jax.lax.bitcast_convert_type(xb, jnp.float32)
    return k(xw, senders.astype(jnp.int32), receivers.astype(jnp.int32))


# speedup vs baseline: 2.1387x; 1.1117x over previous
import functools

import jax
import jax.numpy as jnp
from jax import lax
from jax.experimental import pallas as pl
from jax.experimental.pallas import tpu as pltpu
from jax.experimental.pallas import tpu_sc as plsc

D = 128
L = 16
_UNROLL = 8


def _node_dot_body(x_hbm, s_hbm, r_hbm, out_hbm,
                   s_all, r_all, xs_v, xr_v, o_all, x_sh, sem_s, sem_r,
                   *, n_edges, chunk, num_workers):
    per_w = n_edges // num_workers
    n_chunks = per_w // chunk
    n_groups = chunk // L

    cid = lax.axis_index("c")
    sid = lax.axis_index("s")
    wid = sid * 2 + cid
    base = pl.multiple_of(wid * per_w, chunk)

    iota = lax.iota(jnp.int32, L)

    pltpu.sync_copy(s_hbm.at[pl.ds(base, per_w)], s_all)
    pltpu.sync_copy(r_hbm.at[pl.ds(base, per_w)], r_all)

    @pl.when(sid == 0)
    def _stage():
        pltpu.sync_copy(x_hbm, x_sh)
    plsc.subcore_barrier()

    def start(c, p):
        sl = pl.ds(pl.multiple_of(c * chunk, chunk), chunk)
        pltpu.async_copy(x_sh.at[s_all.at[sl]], xs_v.at[p], sem_s.at[p])
        pltpu.async_copy(x_sh.at[r_all.at[sl]], xr_v.at[p], sem_r.at[p])

    def wait(p):
        pltpu.make_async_copy(x_sh.at[s_all.at[pl.ds(0, chunk)]],
                              xs_v.at[p], sem_s.at[p]).wait()
        pltpu.make_async_copy(x_sh.at[r_all.at[pl.ds(0, chunk)]],
                              xr_v.at[p], sem_r.at[p]).wait()

    def compute(c, p):
        obase = pl.multiple_of(c * chunk, chunk)
        xs_w = xs_v.at[p]
        xr_w = xr_v.at[p]
        dw = D // 2

        def group_body(g, _):
            row = g * L + iota

            def d_body(dd, carry):
                acc, acc2, col = carry
                for _q in range(_UNROLL // 4):
                    mc = None
                    for _j in range(4):
                        a = plsc.load_gather(xs_w, [row, col])
                        b = plsc.load_gather(xr_w, [row, col])
                        m = (plsc.bitcast(a, jnp.bfloat16)
                             * plsc.bitcast(b, jnp.bfloat16))
                        mc = m if mc is None else mc + m
                        col = (col + 1) & (dw - 1)
                    m_lo, m_hi = plsc.unpack(
                        mc, format=plsc.PackFormat.INTERLEAVED)
                    acc = acc + m_lo
                    acc2 = acc2 + m_hi
                return acc, acc2, col

            acc0 = jnp.zeros((L,), jnp.float32)
            acc, acc2, _col = lax.fori_loop(
                0, dw // _UNROLL, d_body, (acc0, acc0, iota))
            o_all[pl.ds(obase + g * L, L)] = acc + acc2
            return 0

        lax.fori_loop(0, n_groups, group_body, 0)

    start(0, 0)
    def pair_body(g, _):
        c0 = g * 2
        start(c0 + 1, 1)
        wait(0)
        compute(c0, 0)
        start(c0 + 2, 0)
        wait(1)
        compute(c0 + 1, 1)
        return 0
    lax.fori_loop(0, (n_chunks - 1) // 2, pair_body, 0)
    wait(0)
    compute(n_chunks - 1, 0)

    pltpu.sync_copy(o_all, out_hbm.at[pl.ds(base, per_w)])


def kernel(x, senders, receivers):
    n_edges = senders.shape[0]
    info = plsc.get_sparse_core_info()
    nw = info.num_cores * info.num_subcores
    chunk = 80
    per_w = n_edges // nw
    assert n_edges % (nw * chunk) == 0 and (per_w // chunk) % 2 == 1

    mesh = plsc.VectorSubcoreMesh(core_axis_name="c", subcore_axis_name="s")
    body = functools.partial(
        _node_dot_body, n_edges=n_edges, chunk=chunk, num_workers=nw)
    k = pl.kernel(
        body,
        out_type=jax.ShapeDtypeStruct((n_edges,), jnp.float32),
        mesh=mesh,
        scratch_types=[
            pltpu.VMEM((per_w,), jnp.int32),
            pltpu.VMEM((per_w,), jnp.int32),
            pltpu.VMEM((2, chunk, D // 2), jnp.float32),
            pltpu.VMEM((2, chunk, D // 2), jnp.float32),
            pltpu.VMEM((per_w,), jnp.float32),
            pltpu.VMEM_SHARED((x.shape[0], D // 2), jnp.float32),
            pltpu.SemaphoreType.DMA((2,)),
            pltpu.SemaphoreType.DMA((2,)),
        ],
        compiler_params=pltpu.CompilerParams(
            needs_layout_passes=False, use_tc_tiling_on_sc=False),
    )
    xb = x.astype(jnp.bfloat16).reshape(x.shape[0], D // 2, 2)
    xw = jax.lax.bitcast_convert_type(xb, jnp.float32)
    return k(xw, senders.astype(jnp.int32), receivers.astype(jnp.int32))
